# Initial kernel scaffold; baseline (speedup 1.0000x reference)
import jax
import jax.numpy as jnp
from jax.experimental import pallas as pl


def kernel(tcword_id, adj, emb, W, a_src, a_dst):
    raise NotImplementedError("write your pallas kernel here")


# speedup vs baseline: 1.6201x; 1.6201x over previous
import functools

import jax
import jax.numpy as jnp
from jax import lax
from jax.experimental import pallas as pl
from jax.experimental.pallas import tpu as pltpu
from jax.experimental.pallas import tpu_sc as plsc

H = 4
O = 32
LEAK = 0.2

_SC_CORES = 2
_SC_SUBCORES = 16
_NW = _SC_CORES * _SC_SUBCORES


def _sc_gather(table, idx):
    V, D = table.shape
    B = idx.shape[0]
    bpw = B // _NW
    mesh = plsc.VectorSubcoreMesh(core_axis_name="c", subcore_axis_name="s")

    @functools.partial(
        pl.kernel,
        mesh=mesh,
        out_type=jax.ShapeDtypeStruct((B, D), jnp.float32),
        scratch_types=[
            pltpu.VMEM((bpw,), jnp.int32),
            pltpu.VMEM((bpw, D), jnp.float32),
            pltpu.SemaphoreType.DMA,
        ],
    )
    def gk(table_hbm, idx_hbm, out_hbm, idx_v, rows_v, sem):
        wid = lax.axis_index("s") * _SC_CORES + lax.axis_index("c")
        base = wid * bpw
        pltpu.sync_copy(idx_hbm.at[pl.ds(base, bpw)], idx_v)
        pltpu.async_copy(table_hbm.at[idx_v], rows_v, sem).wait()
        pltpu.sync_copy(rows_v, out_hbm.at[pl.ds(base, bpw)])

    return gk(table, idx)


def _proj(h, Wfl, Apad):
    Ntot, D = h.shape
    BR = 512

    def body(h_ref, w_ref, a_ref, wh_ref, f_ref):
        wh = jnp.dot(h_ref[...], w_ref[...], preferred_element_type=jnp.float32)
        wh_ref[...] = wh
        f_ref[...] = jnp.dot(wh, a_ref[...], preferred_element_type=jnp.float32)

    return pl.pallas_call(
        body,
        grid=(Ntot // BR,),
        in_specs=[
            pl.BlockSpec((BR, D), lambda i: (i, 0)),
            pl.BlockSpec((D, H * O), lambda i: (0, 0)),
            pl.BlockSpec((D, 16), lambda i: (0, 0)),
        ],
        out_specs=[
            pl.BlockSpec((BR, H * O), lambda i: (i, 0)),
            pl.BlockSpec((BR, 16), lambda i: (i, 0)),
        ],
        out_shape=[
            jax.ShapeDtypeStruct((Ntot, H * O), jnp.float32),
            jax.ShapeDtypeStruct((Ntot, 16), jnp.float32),
        ],
    )(h, Wfl, Apad)


def _pack(F, Ntot):
    fs = F[:, 0:H]
    fd = F[:, 8:8 + H]
    maxd = jnp.max(fd, axis=0, keepdims=True)
    t = fs + maxd
    m = jnp.where(t >= 0, t, LEAK * t)
    srcm = jnp.concatenate([fs, m], axis=1)
    fdstT = jnp.concatenate(
        [fd.T, jnp.zeros((8 - H, Ntot), jnp.float32)], axis=0)
    return srcm, fdstT


def _attn_call(mask, srcm, fdstT, Wh, hres, *, first, last, BI=256, BJ=1024):
    Ntot, Dout = Wh.shape
    NI, NJ = Ntot // BI, Ntot // BJ

    def body(*refs):
        if first:
            (mask_ref, srcm_ref, fdst_ref, wh_ref, hres_ref,
             out_ref, m8_ref, acc_ref, den_ref) = refs
        elif not last:
            (mask_ref, srcm_ref, fdst_ref, wh_ref, hres_ref,
             out_ref, acc_ref, den_ref) = refs
        else:
            (mask_ref, srcm_ref, fdst_ref, wh_ref,
             out_ref, acc_ref, den_ref) = refs
        i = pl.program_id(0)
        j = pl.program_id(1)

        if first:
            rows = i * BI + lax.broadcasted_iota(jnp.int32, (BI, BJ), 0)
            cols = j * BJ + lax.broadcasted_iota(jnp.int32, (BI, BJ), 1)
            keep = jnp.where((mask_ref[...] > 0) | (rows == cols), 1.0, 0.0)
            m8_ref[...] = keep.astype(jnp.int8)
        else:
            keep = mask_ref[...].astype(jnp.float32)

        @pl.when(j == 0)
        def _init():
            acc_ref[...] = jnp.zeros_like(acc_ref)
            den_ref[...] = jnp.zeros_like(den_ref)

        for hh in range(H):
            s = srcm_ref[:, hh:hh + 1]
            m = srcm_ref[:, H + hh:H + hh + 1]
            d = fdst_ref[hh:hh + 1, :]
            t = s + d
            e = jnp.where(t >= 0, t, LEAK * t)
            p = jnp.exp(e - m) * keep
            whj = wh_ref[pl.ds(j * BJ, BJ), O * hh:O * (hh + 1)]
            acc_ref[:, O * hh:O * (hh + 1)] += jnp.dot(
                p, whj, preferred_element_type=jnp.float32)
            den_ref[:, hh:hh + 1] += jnp.sum(p, axis=1, keepdims=True)

        @pl.when(j == NJ - 1)
        def _fin():
            if last:
                tot = acc_ref[:, 0:O] / den_ref[:, 0:1]
                for hh in range(1, H):
                    tot = tot + acc_ref[:, O * hh:O * (hh + 1)] / den_ref[:, hh:hh + 1]
                out_ref[...] = tot * (1.0 / H)
            else:
                for hh in range(H):
                    o = acc_ref[:, O * hh:O * (hh + 1)] / den_ref[:, hh:hh + 1]
                    o = o + hres_ref[:, O * hh:O * (hh + 1)]
                    out_ref[:, O * hh:O * (hh + 1)] = jnp.where(
                        o > 0, o, jnp.exp(o) - 1.0)

    in_specs = [
        pl.BlockSpec((BI, BJ), lambda i, j: (i, j)),
        pl.BlockSpec((BI, 8), lambda i, j: (i, 0)),
        pl.BlockSpec((8, BJ), lambda i, j: (0, j)),
        pl.BlockSpec((Ntot, Dout), lambda i, j: (0, 0)),
    ]
    inputs = [mask, srcm, fdstT, Wh]
    if not last:
        in_specs.append(pl.BlockSpec((BI, Dout), lambda i, j: (i, 0)))
        inputs.append(hres)
        out_specs = [pl.BlockSpec((BI, Dout), lambda i, j: (i, 0))]
        out_shape = [jax.ShapeDtypeStruct((Ntot, Dout), jnp.float32)]
    else:
        out_specs = [pl.BlockSpec((BI, O), lambda i, j: (i, 0))]
        out_shape = [jax.ShapeDtypeStruct((Ntot, O), jnp.float32)]
    if first:
        out_specs.append(pl.BlockSpec((BI, BJ), lambda i, j: (i, j)))
        out_shape.append(jax.ShapeDtypeStruct((Ntot, Ntot), jnp.int8))
    res = pl.pallas_call(
        body,
        grid=(NI, NJ),
        in_specs=in_specs,
        out_specs=out_specs,
        out_shape=out_shape,
        scratch_shapes=[
            pltpu.VMEM((BI, Dout), jnp.float32),
            pltpu.VMEM((BI, 8), jnp.float32),
        ],
        compiler_params=pltpu.CompilerParams(
            dimension_semantics=("arbitrary", "arbitrary")),
    )(*inputs)
    if first:
        return res[0], res[1]
    return res[0]


def kernel(tcword_id, adj, emb, W, a_src, a_dst):
    L = W.shape[0]
    V, D = emb.shape
    Ntot = adj.shape[0]
    idx = tcword_id.astype(jnp.int32)
    h = _sc_gather(emb, idx)

    ind = jnp.kron(jnp.eye(H, dtype=jnp.float32), jnp.ones((O, 1), jnp.float32))
    z4 = jnp.zeros((H * O, H), jnp.float32)
    mask8 = None
    out = None
    for l in range(L):
        Wfl = jnp.transpose(W[l], (1, 0, 2)).reshape(D, H * O)
        Asrc = ind * a_src[l].reshape(-1, 1)
        Adst = ind * a_dst[l].reshape(-1, 1)
        Apad = jnp.concatenate([Asrc, z4, Adst, z4], axis=1)
        Wh, F = _proj(h, Wfl, Apad)
        srcm, fdstT = _pack(F, Ntot)
        if l == 0:
            out, mask8 = _attn_call(adj, srcm, fdstT, Wh, h,
                                    first=True, last=False)
        elif l < L - 1:
            out = _attn_call(mask8, srcm, fdstT, Wh, h,
                             first=False, last=False)
        else:
            out = _attn_call(mask8, srcm, fdstT, Wh, None,
                             first=False, last=True)
        h = out
    return out


# speedup vs baseline: 2.1050x; 1.2993x over previous
import functools

import jax
import jax.numpy as jnp
from jax import lax
from jax.experimental import pallas as pl
from jax.experimental.pallas import tpu as pltpu
from jax.experimental.pallas import tpu_sc as plsc

H = 4
O = 32
LEAK = 0.2

_SC_CORES = 2
_SC_SUBCORES = 16
_NW = _SC_CORES * _SC_SUBCORES


def _sc_gather(table, idx):
    V, D = table.shape
    B = idx.shape[0]
    bpw = B // _NW
    mesh = plsc.VectorSubcoreMesh(core_axis_name="c", subcore_axis_name="s")

    @functools.partial(
        pl.kernel,
        mesh=mesh,
        out_type=jax.ShapeDtypeStruct((B, D), jnp.float32),
        scratch_types=[
            pltpu.VMEM((bpw,), jnp.int32),
            pltpu.VMEM((bpw, D), jnp.float32),
            pltpu.SemaphoreType.DMA,
        ],
    )
    def gk(table_hbm, idx_hbm, out_hbm, idx_v, rows_v, sem):
        wid = lax.axis_index("s") * _SC_CORES + lax.axis_index("c")
        base = wid * bpw
        pltpu.sync_copy(idx_hbm.at[pl.ds(base, bpw)], idx_v)
        pltpu.async_copy(table_hbm.at[idx_v], rows_v, sem).wait()
        pltpu.sync_copy(rows_v, out_hbm.at[pl.ds(base, bpw)])

    return gk(table, idx)


def _proj(h, Waug, Wsmall):
    Ntot, D = h.shape
    BR = 512
    CA = Waug.shape[1]

    def body(h_ref, w_ref, a_ref, wh_ref, f_ref):
        wh = jnp.dot(h_ref[...], w_ref[...], preferred_element_type=jnp.float32)
        col = lax.broadcasted_iota(jnp.int32, (BR, CA), 1)
        wh_ref[...] = jnp.where(col % 64 == O, 1.0, wh)
        f_ref[...] = jnp.dot(h_ref[...], a_ref[...],
                             preferred_element_type=jnp.float32)

    return pl.pallas_call(
        body,
        grid=(Ntot // BR,),
        in_specs=[
            pl.BlockSpec((BR, D), lambda i: (i, 0)),
            pl.BlockSpec((D, CA), lambda i: (0, 0)),
            pl.BlockSpec((D, 16), lambda i: (0, 0)),
        ],
        out_specs=[
            pl.BlockSpec((BR, CA), lambda i: (i, 0)),
            pl.BlockSpec((BR, 16), lambda i: (i, 0)),
        ],
        out_shape=[
            jax.ShapeDtypeStruct((Ntot, CA), jnp.float32),
            jax.ShapeDtypeStruct((Ntot, 16), jnp.float32),
        ],
    )(h, Waug, Wsmall)


def _pack(F, Ntot):
    fs = F[:, 0:H]
    fd = F[:, 8:8 + H]
    maxd = jnp.max(fd, axis=0, keepdims=True)
    t = fs + maxd
    m = jnp.where(t >= 0, t, LEAK * t)
    a1 = jnp.exp(t - m)
    a2 = jnp.exp(LEAK * t - m)
    u = fd - maxd
    b1 = jnp.exp(u)
    b2 = jnp.exp(LEAK * u)
    z = jnp.zeros((Ntot, H), jnp.float32)
    srcm = jnp.concatenate([-fs, a1, a2, z], axis=1)
    fdstT = jnp.concatenate([fd.T, b1.T, b2.T, z.T], axis=0)
    return srcm, fdstT


def _attn_call(mask, srcm, fdstT, Wh, hres, *, first, last, BI=256, BJ=1024):
    Ntot = Wh.shape[0]
    Dout = H * O
    NI, NJ = Ntot // BI, Ntot // BJ

    def body(*refs):
        if first:
            (mask_ref, srcm_ref, fdst_ref, wh_ref, hres_ref,
             out_ref, m8_ref, acc_ref) = refs
        elif not last:
            (mask_ref, srcm_ref, fdst_ref, wh_ref, hres_ref,
             out_ref, acc_ref) = refs
        else:
            (mask_ref, srcm_ref, fdst_ref, wh_ref,
             out_ref, acc_ref) = refs
        i = pl.program_id(0)
        j = pl.program_id(1)

        if first:
            rows = i * BI + lax.broadcasted_iota(jnp.int32, (BI, BJ), 0)
            cols = j * BJ + lax.broadcasted_iota(jnp.int32, (BI, BJ), 1)
            keep = jnp.where((mask_ref[...] > 0) | (rows == cols), 1.0, 0.0)
            m8_ref[...] = keep.astype(jnp.int8)
        else:
            keep = mask_ref[...].astype(jnp.float32)

        @pl.when(j == 0)
        def _init():
            acc_ref[...] = jnp.zeros_like(acc_ref)

        for hh in range(H):
            ns = srcm_ref[:, hh:hh + 1]
            a1 = srcm_ref[:, H + hh:H + hh + 1]
            a2 = srcm_ref[:, 2 * H + hh:2 * H + hh + 1]
            d = fdst_ref[hh:hh + 1, :]
            b1 = fdst_ref[H + hh:H + hh + 1, :]
            b2 = fdst_ref[2 * H + hh:2 * H + hh + 1, :]
            p = jnp.where(d >= ns, a1 * b1, a2 * b2) * keep
            whj = wh_ref[pl.ds(j * BJ, BJ), 64 * hh:64 * (hh + 1)]
            acc_ref[:, 64 * hh:64 * (hh + 1)] += jnp.dot(
                p, whj, preferred_element_type=jnp.float32)

        @pl.when(j == NJ - 1)
        def _fin():
            if last:
                tot = acc_ref[:, 0:O] / acc_ref[:, O:O + 1]
                for hh in range(1, H):
                    tot = tot + (acc_ref[:, 64 * hh:64 * hh + O]
                                 / acc_ref[:, 64 * hh + O:64 * hh + O + 1])
                out_ref[...] = tot * (1.0 / H)
            else:
                for hh in range(H):
                    o = (acc_ref[:, 64 * hh:64 * hh + O]
                         / acc_ref[:, 64 * hh + O:64 * hh + O + 1])
                    o = o + hres_ref[:, O * hh:O * (hh + 1)]
                    out_ref[:, O * hh:O * (hh + 1)] = jnp.where(
                        o > 0, o, jnp.exp(o) - 1.0)

    in_specs = [
        pl.BlockSpec((BI, BJ), lambda i, j: (i, j)),
        pl.BlockSpec((BI, 16), lambda i, j: (i, 0)),
        pl.BlockSpec((16, BJ), lambda i, j: (0, j)),
        pl.BlockSpec((Ntot, 64 * H), lambda i, j: (0, 0)),
    ]
    inputs = [mask, srcm, fdstT, Wh]
    if not last:
        in_specs.append(pl.BlockSpec((BI, Dout), lambda i, j: (i, 0)))
        inputs.append(hres)
        out_specs = [pl.BlockSpec((BI, Dout), lambda i, j: (i, 0))]
        out_shape = [jax.ShapeDtypeStruct((Ntot, Dout), jnp.float32)]
    else:
        out_specs = [pl.BlockSpec((BI, O), lambda i, j: (i, 0))]
        out_shape = [jax.ShapeDtypeStruct((Ntot, O), jnp.float32)]
    if first:
        out_specs.append(pl.BlockSpec((BI, BJ), lambda i, j: (i, j)))
        out_shape.append(jax.ShapeDtypeStruct((Ntot, Ntot), jnp.int8))
    res = pl.pallas_call(
        body,
        grid=(NI, NJ),
        in_specs=in_specs,
        out_specs=out_specs,
        out_shape=out_shape,
        scratch_shapes=[
            pltpu.VMEM((BI, 64 * H), jnp.float32),
        ],
        compiler_params=pltpu.CompilerParams(
            dimension_semantics=("arbitrary", "arbitrary")),
    )(*inputs)
    if first:
        return res[0], res[1]
    return res[0]


def kernel(tcword_id, adj, emb, W, a_src, a_dst):
    L = W.shape[0]
    V, D = emb.shape
    Ntot = adj.shape[0]
    idx = tcword_id.astype(jnp.int32)
    h = _sc_gather(emb, idx)

    ind = jnp.kron(jnp.eye(H, dtype=jnp.float32), jnp.ones((O, 1), jnp.float32))
    z4 = jnp.zeros((H * O, H), jnp.float32)
    mask8 = None
    out = None
    for l in range(L):
        Wfl = jnp.transpose(W[l], (1, 0, 2)).reshape(D, H * O)
        Asrc = ind * a_src[l].reshape(-1, 1)
        Adst = ind * a_dst[l].reshape(-1, 1)
        Apad = jnp.concatenate([Asrc, z4, Adst, z4], axis=1)
        Wsmall = Wfl @ Apad
        Waug = jnp.zeros((D, 64 * H), jnp.float32)
        for hh in range(H):
            Waug = Waug.at[:, 64 * hh:64 * hh + O].set(Wfl[:, O * hh:O * (hh + 1)])
        Wh, F = _proj(h, Waug, Wsmall)
        srcm, fdstT = _pack(F, Ntot)
        if l == 0:
            out, mask8 = _attn_call(adj, srcm, fdstT, Wh, h,
                                    first=True, last=False)
        elif l < L - 1:
            out = _attn_call(mask8, srcm, fdstT, Wh, h,
                             first=False, last=False)
        else:
            out = _attn_call(mask8, srcm, fdstT, Wh, None,
                             first=False, last=True)
        h = out
    return out


# speedup vs baseline: 3.1041x; 1.4747x over previous
import functools

import jax
import jax.numpy as jnp
from jax import lax
from jax.experimental import pallas as pl
from jax.experimental.pallas import tpu as pltpu
from jax.experimental.pallas import tpu_sc as plsc

H = 4
O = 32
LEAK = 0.2

_SC_CORES = 2
_SC_SUBCORES = 16
_NW = _SC_CORES * _SC_SUBCORES


def _sc_gather(table, idx):
    V, D = table.shape
    B = idx.shape[0]
    bpw = B // _NW
    mesh = plsc.VectorSubcoreMesh(core_axis_name="c", subcore_axis_name="s")

    @functools.partial(
        pl.kernel,
        mesh=mesh,
        out_type=jax.ShapeDtypeStruct((B, D), jnp.float32),
        scratch_types=[
            pltpu.VMEM((bpw,), jnp.int32),
            pltpu.VMEM((bpw, D), jnp.float32),
            pltpu.SemaphoreType.DMA,
        ],
    )
    def gk(table_hbm, idx_hbm, out_hbm, idx_v, rows_v, sem):
        wid = lax.axis_index("s") * _SC_CORES + lax.axis_index("c")
        base = wid * bpw
        pltpu.sync_copy(idx_hbm.at[pl.ds(base, bpw)], idx_v)
        pltpu.async_copy(table_hbm.at[idx_v], rows_v, sem).wait()
        pltpu.sync_copy(rows_v, out_hbm.at[pl.ds(base, bpw)])

    return gk(table, idx)


def _proj(h, Waug, Wsmall):
    Ntot, D = h.shape
    BR = 512
    CA = Waug.shape[1]

    def body(h_ref, w_ref, a_ref, wh_ref, f_ref):
        wh = jnp.dot(h_ref[...], w_ref[...], preferred_element_type=jnp.float32)
        col = lax.broadcasted_iota(jnp.int32, (BR, CA), 1)
        wh_ref[...] = jnp.where(col % 64 == O, 1.0, wh)
        f_ref[...] = jnp.dot(h_ref[...], a_ref[...],
                             preferred_element_type=jnp.float32)

    return pl.pallas_call(
        body,
        grid=(Ntot // BR,),
        in_specs=[
            pl.BlockSpec((BR, D), lambda i: (i, 0)),
            pl.BlockSpec((D, CA), lambda i: (0, 0)),
            pl.BlockSpec((D, 16), lambda i: (0, 0)),
        ],
        out_specs=[
            pl.BlockSpec((BR, CA), lambda i: (i, 0)),
            pl.BlockSpec((BR, 16), lambda i: (i, 0)),
        ],
        out_shape=[
            jax.ShapeDtypeStruct((Ntot, CA), jnp.float32),
            jax.ShapeDtypeStruct((Ntot, 16), jnp.float32),
        ],
    )(h, Waug, Wsmall)


def _pack(F, Ntot):
    fs = F[:, 0:H]
    fd = F[:, 8:8 + H]
    maxd = jnp.max(fd, axis=0, keepdims=True)
    t = fs + maxd
    m = jnp.where(t >= 0, t, LEAK * t)
    a1 = jnp.exp(t - m)
    a2 = jnp.exp(LEAK * t - m)
    u = fd - maxd
    b1 = jnp.exp(u)
    b2 = jnp.exp(LEAK * u)
    srcm = jnp.concatenate([a1, a2], axis=1)
    fdstT = jnp.concatenate([b1.T, b2.T], axis=0)
    return srcm, fdstT


def _attn_call(mask, srcm, fdstT, Wh, hres, *, first, last, BI=512, BJ=2048):
    Ntot = Wh.shape[0]
    Dout = H * O
    NI, NJ = Ntot // BI, Ntot // BJ

    def body(*refs):
        if first:
            (mask_ref, srcm_ref, fdst_ref, wh_ref, hres_ref,
             out_ref, m8_ref, acc_ref) = refs
        elif not last:
            (mask_ref, srcm_ref, fdst_ref, wh_ref, hres_ref,
             out_ref, acc_ref) = refs
        else:
            (mask_ref, srcm_ref, fdst_ref, wh_ref,
             out_ref, acc_ref) = refs
        i = pl.program_id(0)
        j = pl.program_id(1)

        if first:
            rows = i * BI + lax.broadcasted_iota(jnp.int32, (BI, BJ), 0)
            cols = j * BJ + lax.broadcasted_iota(jnp.int32, (BI, BJ), 1)
            keep = jnp.where((mask_ref[...] > 0) | (rows == cols), 1.0, 0.0)
            m8_ref[...] = keep.astype(jnp.int8)
        else:
            keep = mask_ref[...].astype(jnp.float32)

        @pl.when(j == 0)
        def _init():
            acc_ref[...] = jnp.zeros_like(acc_ref)

        for hh in range(H):
            a1 = srcm_ref[:, hh:hh + 1]
            a2 = srcm_ref[:, H + hh:H + hh + 1]
            b1 = fdst_ref[hh:hh + 1, :]
            b2 = fdst_ref[H + hh:H + hh + 1, :]
            p = jnp.maximum(a1 * b1, a2 * b2) * keep
            whj = wh_ref[pl.ds(j * BJ, BJ), 64 * hh:64 * (hh + 1)]
            acc_ref[:, 64 * hh:64 * (hh + 1)] += jnp.dot(
                p, whj, preferred_element_type=jnp.float32)

        @pl.when(j == NJ - 1)
        def _fin():
            if last:
                tot = acc_ref[:, 0:O] / acc_ref[:, O:O + 1]
                for hh in range(1, H):
                    tot = tot + (acc_ref[:, 64 * hh:64 * hh + O]
                                 / acc_ref[:, 64 * hh + O:64 * hh + O + 1])
                out_ref[...] = tot * (1.0 / H)
            else:
                for hh in range(H):
                    o = (acc_ref[:, 64 * hh:64 * hh + O]
                         / acc_ref[:, 64 * hh + O:64 * hh + O + 1])
                    o = o + hres_ref[:, O * hh:O * (hh + 1)]
                    out_ref[:, O * hh:O * (hh + 1)] = jnp.where(
                        o > 0, o, jnp.exp(o) - 1.0)

    in_specs = [
        pl.BlockSpec((BI, BJ), lambda i, j: (i, j)),
        pl.BlockSpec((BI, 8), lambda i, j: (i, 0)),
        pl.BlockSpec((8, BJ), lambda i, j: (0, j)),
        pl.BlockSpec((Ntot, 64 * H), lambda i, j: (0, 0)),
    ]
    inputs = [mask, srcm, fdstT, Wh]
    if not last:
        in_specs.append(pl.BlockSpec((BI, Dout), lambda i, j: (i, 0)))
        inputs.append(hres)
        out_specs = [pl.BlockSpec((BI, Dout), lambda i, j: (i, 0))]
        out_shape = [jax.ShapeDtypeStruct((Ntot, Dout), jnp.float32)]
    else:
        out_specs = [pl.BlockSpec((BI, O), lambda i, j: (i, 0))]
        out_shape = [jax.ShapeDtypeStruct((Ntot, O), jnp.float32)]
    if first:
        out_specs.append(pl.BlockSpec((BI, BJ), lambda i, j: (i, j)))
        out_shape.append(jax.ShapeDtypeStruct((Ntot, Ntot), jnp.int8))
    res = pl.pallas_call(
        body,
        grid=(NI, NJ),
        in_specs=in_specs,
        out_specs=out_specs,
        out_shape=out_shape,
        scratch_shapes=[
            pltpu.VMEM((BI, 64 * H), jnp.float32),
        ],
        compiler_params=pltpu.CompilerParams(
            dimension_semantics=("arbitrary", "arbitrary")),
    )(*inputs)
    if first:
        return res[0], res[1]
    return res[0]


def kernel(tcword_id, adj, emb, W, a_src, a_dst):
    L = W.shape[0]
    V, D = emb.shape
    Ntot = adj.shape[0]
    idx = tcword_id.astype(jnp.int32)
    h = _sc_gather(emb, idx)

    ind = jnp.kron(jnp.eye(H, dtype=jnp.float32), jnp.ones((O, 1), jnp.float32))
    z4 = jnp.zeros((H * O, H), jnp.float32)
    mask8 = None
    out = None
    for l in range(L):
        Wfl = jnp.transpose(W[l], (1, 0, 2)).reshape(D, H * O)
        Asrc = ind * a_src[l].reshape(-1, 1)
        Adst = ind * a_dst[l].reshape(-1, 1)
        Apad = jnp.concatenate([Asrc, z4, Adst, z4], axis=1)
        Wsmall = Wfl @ Apad
        Waug = jnp.zeros((D, 64 * H), jnp.float32)
        for hh in range(H):
            Waug = Waug.at[:, 64 * hh:64 * hh + O].set(Wfl[:, O * hh:O * (hh + 1)])
        Wh, F = _proj(h, Waug, Wsmall)
        srcm, fdstT = _pack(F, Ntot)
        if l == 0:
            out, mask8 = _attn_call(adj, srcm, fdstT, Wh, h,
                                    first=True, last=False)
        elif l < L - 1:
            out = _attn_call(mask8, srcm, fdstT, Wh, h,
                             first=False, last=False)
        else:
            out = _attn_call(mask8, srcm, fdstT, Wh, None,
                             first=False, last=True)
        h = out
    return out


# speedup vs baseline: 3.2589x; 1.0499x over previous
import functools

import jax
import jax.numpy as jnp
from jax import lax
from jax.experimental import pallas as pl
from jax.experimental.pallas import tpu as pltpu
from jax.experimental.pallas import tpu_sc as plsc

H = 4
O = 32
LEAK = 0.2

_SC_CORES = 2
_SC_SUBCORES = 16
_NW = _SC_CORES * _SC_SUBCORES


def _sc_gather(table, idx):
    V, D = table.shape
    B = idx.shape[0]
    bpw = B // _NW
    mesh = plsc.VectorSubcoreMesh(core_axis_name="c", subcore_axis_name="s")

    @functools.partial(
        pl.kernel,
        mesh=mesh,
        out_type=jax.ShapeDtypeStruct((B, D), jnp.float32),
        scratch_types=[
            pltpu.VMEM((bpw,), jnp.int32),
            pltpu.VMEM((bpw, D), jnp.float32),
            pltpu.SemaphoreType.DMA,
        ],
    )
    def gk(table_hbm, idx_hbm, out_hbm, idx_v, rows_v, sem):
        wid = lax.axis_index("s") * _SC_CORES + lax.axis_index("c")
        base = wid * bpw
        pltpu.sync_copy(idx_hbm.at[pl.ds(base, bpw)], idx_v)
        pltpu.async_copy(table_hbm.at[idx_v], rows_v, sem).wait()
        pltpu.sync_copy(rows_v, out_hbm.at[pl.ds(base, bpw)])

    return gk(table, idx)


def _proj(h, Waug, Wsmall):
    Ntot, D = h.shape
    BR = 512
    CA = Waug.shape[1]

    def body(h_ref, w_ref, a_ref, wh_ref, f_ref):
        wh = jnp.dot(h_ref[...], w_ref[...], preferred_element_type=jnp.float32)
        col = lax.broadcasted_iota(jnp.int32, (BR, CA), 1)
        wh_ref[...] = jnp.where(col % 64 == O, 1.0, wh)
        f_ref[...] = jnp.dot(h_ref[...], a_ref[...],
                             preferred_element_type=jnp.float32)

    return pl.pallas_call(
        body,
        grid=(Ntot // BR,),
        in_specs=[
            pl.BlockSpec((BR, D), lambda i: (i, 0)),
            pl.BlockSpec((D, CA), lambda i: (0, 0)),
            pl.BlockSpec((D, 16), lambda i: (0, 0)),
        ],
        out_specs=[
            pl.BlockSpec((BR, CA), lambda i: (i, 0)),
            pl.BlockSpec((BR, 16), lambda i: (i, 0)),
        ],
        out_shape=[
            jax.ShapeDtypeStruct((Ntot, CA), jnp.float32),
            jax.ShapeDtypeStruct((Ntot, 16), jnp.float32),
        ],
    )(h, Waug, Wsmall)


def _pack(F, Ntot):
    fs = F[:, 0:H]
    fd = F[:, 8:8 + H]
    maxd = jnp.max(fd, axis=0, keepdims=True)
    t = fs + maxd
    m = jnp.where(t >= 0, t, LEAK * t)
    a1 = jnp.exp(t - m)
    a2 = jnp.exp(LEAK * t - m)
    u = fd - maxd
    b1 = jnp.exp(u)
    b2 = jnp.exp(LEAK * u)
    srcm = jnp.concatenate([a1, a2], axis=1)
    fdstT = jnp.concatenate([b1.T, b2.T], axis=0)
    return srcm, fdstT


def _attn_call(mask, srcm, fdstT, Wh, hres, *, first, last, BI=512, BJ=2048):
    Ntot = Wh.shape[0]
    Dout = H * O
    NI, NJ = Ntot // BI, Ntot // BJ

    def body(*refs):
        if first:
            (mask_ref, srcm_ref, fdst_ref, wh_ref, hres_ref,
             out_ref, m8_ref, acc_ref) = refs
        elif not last:
            (mask_ref, srcm_ref, fdst_ref, wh_ref, hres_ref,
             out_ref, acc_ref) = refs
        else:
            (mask_ref, srcm_ref, fdst_ref, wh_ref,
             out_ref, acc_ref) = refs
        i = pl.program_id(0)
        j = pl.program_id(1)

        if first:
            rows = i * BI + lax.broadcasted_iota(jnp.int32, (BI, BJ), 0)
            cols = j * BJ + lax.broadcasted_iota(jnp.int32, (BI, BJ), 1)
            keep = jnp.where((mask_ref[...] > 0) | (rows == cols), 1.0, 0.0)
            m8_ref[...] = keep.astype(jnp.int8)
        else:
            keep = mask_ref[...].astype(jnp.float32)

        @pl.when(j == 0)
        def _init():
            acc_ref[...] = jnp.zeros_like(acc_ref)

        for hh in range(H):
            a1 = srcm_ref[:, hh:hh + 1]
            a2 = srcm_ref[:, H + hh:H + hh + 1]
            b1 = fdst_ref[hh:hh + 1, :]
            b2 = fdst_ref[H + hh:H + hh + 1, :]
            p = jnp.maximum(a1 * b1, a2 * b2) * keep
            whj = wh_ref[pl.ds(j * BJ, BJ), 64 * hh:64 * (hh + 1)]
            acc_ref[:, 64 * hh:64 * (hh + 1)] += jnp.dot(
                p, whj, preferred_element_type=jnp.float32)

        @pl.when(j == NJ - 1)
        def _fin():
            if last:
                tot = acc_ref[:, 0:O] / acc_ref[:, O:O + 1]
                for hh in range(1, H):
                    tot = tot + (acc_ref[:, 64 * hh:64 * hh + O]
                                 / acc_ref[:, 64 * hh + O:64 * hh + O + 1])
                out_ref[...] = tot * (1.0 / H)
            else:
                for hh in range(H):
                    o = (acc_ref[:, 64 * hh:64 * hh + O]
                         / acc_ref[:, 64 * hh + O:64 * hh + O + 1])
                    o = o + hres_ref[:, O * hh:O * (hh + 1)]
                    out_ref[:, O * hh:O * (hh + 1)] = jnp.where(
                        o > 0, o, jnp.exp(o) - 1.0)

    in_specs = [
        pl.BlockSpec((BI, BJ), lambda i, j: (i, j)),
        pl.BlockSpec((BI, 8), lambda i, j: (i, 0)),
        pl.BlockSpec((8, BJ), lambda i, j: (0, j)),
        pl.BlockSpec((Ntot, 64 * H), lambda i, j: (0, 0)),
    ]
    inputs = [mask, srcm, fdstT, Wh]
    if not last:
        in_specs.append(pl.BlockSpec((BI, Dout), lambda i, j: (i, 0)))
        inputs.append(hres)
        out_specs = [pl.BlockSpec((BI, Dout), lambda i, j: (i, 0))]
        out_shape = [jax.ShapeDtypeStruct((Ntot, Dout), jnp.float32)]
    else:
        out_specs = [pl.BlockSpec((BI, O), lambda i, j: (i, 0))]
        out_shape = [jax.ShapeDtypeStruct((Ntot, O), jnp.float32)]
    if first:
        out_specs.append(pl.BlockSpec((BI, BJ), lambda i, j: (i, j)))
        out_shape.append(jax.ShapeDtypeStruct((Ntot, Ntot), jnp.int8))
    res = pl.pallas_call(
        body,
        grid=(NI, NJ),
        in_specs=in_specs,
        out_specs=out_specs,
        out_shape=out_shape,
        scratch_shapes=[
            pltpu.VMEM((BI, 64 * H), jnp.float32),
        ],
        compiler_params=pltpu.CompilerParams(
            dimension_semantics=("arbitrary", "arbitrary")),
    )(*inputs)
    if first:
        return res[0], res[1]
    return res[0]


def kernel(tcword_id, adj, emb, W, a_src, a_dst):
    L = W.shape[0]
    V, D = emb.shape
    Ntot = adj.shape[0]
    idx = tcword_id.astype(jnp.int32)
    h = _sc_gather(emb, idx)

    Wt = jnp.transpose(W, (0, 2, 1, 3))
    Waug_all = jnp.pad(Wt, ((0, 0), (0, 0), (0, 0), (0, 64 - O))
                       ).reshape(L, D, 64 * H)
    Wfl_all = Wt.reshape(L, D, H * O)
    ind = jnp.kron(jnp.eye(H, dtype=jnp.float32), jnp.ones((O, 1), jnp.float32))
    zL = jnp.zeros((L, H * O, H), jnp.float32)
    Asrc_all = ind[None] * a_src.reshape(L, H * O, 1)
    Adst_all = ind[None] * a_dst.reshape(L, H * O, 1)
    Apad_all = jnp.concatenate([Asrc_all, zL, Adst_all, zL], axis=2)
    Wsmall_all = jnp.einsum('lde,lef->ldf', Wfl_all, Apad_all)

    mask8 = None
    out = None
    for l in range(L):
        Wh, F = _proj(h, Waug_all[l], Wsmall_all[l])
        srcm, fdstT = _pack(F, Ntot)
        if l == 0:
            out, mask8 = _attn_call(adj, srcm, fdstT, Wh, h,
                                    first=True, last=False)
        elif l < L - 1:
            out = _attn_call(mask8, srcm, fdstT, Wh, h,
                             first=False, last=False)
        else:
            out = _attn_call(mask8, srcm, fdstT, Wh, None,
                             first=False, last=True)
        h = out
    return out
